# baseline (speedup 1.0000x reference)
import functools

import jax
import jax.numpy as jnp
from jax.experimental import pallas as pl
from jax.experimental.pallas import tpu as pltpu

B, T, N, C = 64, 256, 21, 3
HID, HEADS, HDIM, DOUT = 128, 4, 32, 256
CDIM, DMODEL = 64, 256
DINNER, DSTATE, DCONV, DTRANK, L = 512, 16, 4, 16, 4
NSIGNS, NHS, NLOC, NMOV, NORI = 2000, 40, 20, 20, 8
TEMP = 0.07

F = 32
BB = 4
TC = 8


def _agan_body(x_ref, w_in_ref, b_in_ref, wgat_ref, ssrc_ref, sdst_ref,
               w_out_ref, b_out_ref, pdmw_ref, pdmb_ref, wfuse_ref, bfuse_ref,
               fused_ref, csum_ref):
    tb = pl.program_id(1)
    xb = x_ref[...]
    h0 = jnp.maximum(jnp.dot(xb, w_in_ref[...]) + b_in_ref[...], 0.0)
    hf = jnp.dot(h0, wgat_ref[...])
    src = jnp.dot(hf, ssrc_ref[...])
    dst = jnp.dot(hf, sdst_ref[...])
    src3 = src.reshape(F, N, HEADS * HDIM)
    dst3 = dst.reshape(F, N, HEADS * HDIM)
    e = src3[:, :, None, :] + dst3[:, None, :, :]
    e = jnp.where(e >= 0.0, e, 0.2 * e)
    m = jnp.max(e, axis=2, keepdims=True)
    ex = jnp.exp(e - m)
    alpha = ex / jnp.sum(ex, axis=2, keepdims=True)
    hf3 = hf.reshape(F, N, HEADS * HDIM)
    agg = jnp.sum(alpha * hf3[:, None, :, :], axis=2)
    feat = jnp.maximum(h0.reshape(F, N, HID) + agg, 0.0)
    node = jnp.dot(feat.reshape(F * N, HID), w_out_ref[...]) + b_out_ref[...]
    spatial = jnp.mean(node.reshape(F, N, DOUT), axis=1)
    comps = jnp.maximum(jnp.dot(spatial, pdmw_ref[...]) + pdmb_ref[...], 0.0)
    fused = jnp.dot(comps, wfuse_ref[...]) + bfuse_ref[...]
    fused_ref[...] = fused[None]
    part = jnp.sum(comps, axis=0, keepdims=True)[None]

    @pl.when(tb == 0)
    def _():
        csum_ref[...] = part

    @pl.when(tb > 0)
    def _():
        csum_ref[...] += part


def _ssm_body(fused_ref, lnw_ref, ipr_ref, cwt_ref, cb_ref, xpr_ref,
              dtw_ref, dtb_ref, alogT_ref, d_ref, opr_ref, tmean_ref,
              dt_s, du_s, y_s, bmT_s, cmT_s):
    xx2 = fused_ref[...].reshape(BB * T, DMODEL)
    for l in range(L):
        mu = jnp.mean(xx2, axis=-1, keepdims=True)
        var = jnp.mean((xx2 - mu) ** 2, axis=-1, keepdims=True)
        xn2 = (xx2 - mu) * jax.lax.rsqrt(var + 1e-5) * lnw_ref[l][None, :]
        ysum2 = jnp.zeros((BB * T, DMODEL), jnp.float32)
        for d in range(2):
            fwd = (d == 0)
            xz2 = jnp.dot(xn2, ipr_ref[l, d])
            u3 = xz2[:, :DINNER].reshape(BB, T, DINNER)
            z2 = xz2[:, DINNER:]
            acc = jnp.zeros((BB, T, DINNER), jnp.float32)
            for k in range(DCONV):
                w_k = cwt_ref[l, d, k][None, None, :]
                sh = DCONV - 1 - k
                if sh == 0:
                    term = u3
                elif fwd:
                    term = jnp.concatenate(
                        [jnp.zeros((BB, sh, DINNER), jnp.float32),
                         u3[:, :T - sh, :]], axis=1)
                else:
                    term = jnp.concatenate(
                        [u3[:, sh:, :],
                         jnp.zeros((BB, sh, DINNER), jnp.float32)], axis=1)
                acc = acc + term * w_k
            uc3 = acc + cb_ref[l, d][None, None, :]
            uc3 = uc3 * jax.nn.sigmoid(uc3)
            uc2 = uc3.reshape(BB * T, DINNER)
            xdbl = jnp.dot(uc2, xpr_ref[l, d])
            dt2 = jax.nn.softplus(
                jnp.dot(xdbl[:, :DTRANK], dtw_ref[l, d]) + dtb_ref[l, d][None, :])
            dt_s[...] = dt2.reshape(BB, T, DINNER)
            du_s[...] = (dt2 * uc2).reshape(BB, T, DINNER)
            bmT_s[...] = jnp.transpose(
                xdbl[:, DTRANK:DTRANK + DSTATE].reshape(BB, T // TC, TC, DSTATE),
                (1, 0, 2, 3))
            cmT_s[...] = jnp.transpose(
                xdbl[:, DTRANK + DSTATE:].reshape(BB, T // TC, TC, DSTATE),
                (1, 0, 2, 3))
            at = -jnp.exp(alogT_ref[l, d])

            def chunk(c, h, _fwd=fwd, _at=at):
                t0 = c * TC if _fwd else (T - TC) - c * TC
                ci = c if _fwd else (T // TC - 1) - c
                dtc = dt_s[:, pl.ds(t0, TC), :]
                duc = du_s[:, pl.ds(t0, TC), :]
                bc = bmT_s[ci]
                cc = cmT_s[ci]
                dAc = jnp.exp(dtc[:, :, None, :] * _at[None, None])
                dBc = bc[:, :, :, None] * duc[:, :, None, :]
                hs = [None] * TC
                order = range(TC) if _fwd else range(TC - 1, -1, -1)
                for j in order:
                    h = dAc[:, j] * h + dBc[:, j]
                    hs[j] = h
                hcat = jnp.stack(hs, axis=1)
                y_s[:, pl.ds(t0, TC), :] = jnp.sum(
                    hcat * cc[:, :, :, None], axis=2)
                return h

            h0 = jnp.zeros((BB, DSTATE, DINNER), jnp.float32)
            jax.lax.fori_loop(0, T // TC, chunk, h0)
            y3 = y_s[...] + uc3 * d_ref[l, d][None, None, :]
            g2 = (y3.reshape(BB * T, DINNER)) * (z2 * jax.nn.sigmoid(z2))
            ysum2 = ysum2 + jnp.dot(g2, opr_ref[l, d])
        xx2 = xx2 + ysum2
    tmean_ref[...] = jnp.mean(xx2.reshape(BB, T, DMODEL), axis=1,
                              keepdims=True)


def _cosn(f):
    return f / (jnp.sqrt(jnp.sum(f * f, axis=-1, keepdims=True)) + 1e-8)


def _hpc_body(tm_ref, w_ref, b_ref, sp_ref, cs_ref, phs_ref, plo_ref,
              pmv_ref, por_ref, logits_ref, emb_ref, s1_ref, s2_ref,
              s3_ref, s4_ref):
    emb = jnp.dot(tm_ref[...], w_ref[...]) + b_ref[...]
    emb_ref[...] = emb
    en = _cosn(emb)
    pn = _cosn(sp_ref[...])
    logits_ref[...] = jax.lax.dot_general(
        en, pn, (((1,), (1,)), ((), ()))) * (1.0 / TEMP)
    pooled = cs_ref[...] * (1.0 / T)
    for k, (p_ref, o_ref) in enumerate(((phs_ref, s1_ref), (plo_ref, s2_ref),
                                        (pmv_ref, s3_ref), (por_ref, s4_ref))):
        fn = _cosn(pooled[:, k * CDIM:(k + 1) * CDIM])
        o_ref[...] = jax.lax.dot_general(
            fn, _cosn(p_ref[...]), (((1,), (1,)), ((), ()))) * (1.0 / TEMP)


def _full(spec_shape, ndim=None):
    nd = len(spec_shape)
    return pl.BlockSpec(spec_shape, lambda *_: (0,) * nd)


def kernel(x, agan_w_in, agan_b_in, agan_w_gat, agan_a_src, agan_a_dst,
           agan_w_out, agan_b_out, pdm_w, pdm_b, pdm_w_fuse, pdm_b_fuse,
           ssm_ln_w, ssm_in_proj, ssm_conv_w, ssm_conv_b, ssm_x_proj,
           ssm_dt_w, ssm_dt_b, ssm_A_log, ssm_D, ssm_out_proj,
           hpc_w_sign, hpc_b_sign, sign_protos, proto_hs, proto_loc,
           proto_mov, proto_ori):
    f32 = jnp.float32
    xr = x.reshape(B * T * N, C)
    wgat = agan_w_gat.transpose(1, 0, 2).reshape(HID, HEADS * HDIM)
    headmask = jnp.kron(jnp.eye(HEADS, dtype=f32),
                        jnp.ones((HDIM, HDIM), f32))
    ssrc = agan_a_src.reshape(-1)[:, None] * headmask
    sdst = agan_a_dst.reshape(-1)[:, None] * headmask
    pdmw = pdm_w.transpose(1, 0, 2).reshape(DOUT, 4 * CDIM)
    pdmb = pdm_b.reshape(1, 4 * CDIM)

    nf = T // F
    fused, csum = pl.pallas_call(
        _agan_body,
        grid=(B, nf),
        in_specs=[
            pl.BlockSpec((F * N, C), lambda b, t: (b * nf + t, 0)),
            _full((C, HID)), _full((1, HID)), _full((HID, HEADS * HDIM)),
            _full((HID, HID)), _full((HID, HID)),
            _full((HID, DOUT)), _full((1, DOUT)),
            _full((DOUT, 4 * CDIM)), _full((1, 4 * CDIM)),
            _full((4 * CDIM, DMODEL)), _full((1, DMODEL)),
        ],
        out_specs=[
            pl.BlockSpec((1, F, DMODEL), lambda b, t: (b, t, 0)),
            pl.BlockSpec((1, 1, DMODEL), lambda b, t: (b, 0, 0)),
        ],
        out_shape=[
            jax.ShapeDtypeStruct((B, T, DMODEL), f32),
            jax.ShapeDtypeStruct((B, 1, DMODEL), f32),
        ],
        compiler_params=pltpu.CompilerParams(
            dimension_semantics=("parallel", "arbitrary"),
            vmem_limit_bytes=100 * 1024 * 1024,
        ),
    )(xr, agan_w_in, agan_b_in.reshape(1, HID), wgat, ssrc, sdst,
      agan_w_out, agan_b_out.reshape(1, DOUT), pdmw, pdmb,
      pdm_w_fuse, pdm_b_fuse.reshape(1, DMODEL))

    cwt = ssm_conv_w.transpose(0, 1, 3, 2)
    alogT = ssm_A_log.transpose(0, 1, 3, 2)

    tmean = pl.pallas_call(
        _ssm_body,
        grid=(B // BB,),
        in_specs=[
            pl.BlockSpec((BB, T, DMODEL), lambda i: (i, 0, 0)),
            _full((L, DMODEL)),
            _full((L, 2, DMODEL, 2 * DINNER)),
            _full((L, 2, DCONV, DINNER)),
            _full((L, 2, DINNER)),
            _full((L, 2, DINNER, DTRANK + 2 * DSTATE)),
            _full((L, 2, DTRANK, DINNER)),
            _full((L, 2, DINNER)),
            _full((L, 2, DSTATE, DINNER)),
            _full((L, 2, DINNER)),
            _full((L, 2, DINNER, DMODEL)),
        ],
        out_specs=pl.BlockSpec((BB, 1, DMODEL), lambda i: (i, 0, 0)),
        out_shape=jax.ShapeDtypeStruct((B, 1, DMODEL), f32),
        scratch_shapes=[
            pltpu.VMEM((BB, T, DINNER), f32),
            pltpu.VMEM((BB, T, DINNER), f32),
            pltpu.VMEM((BB, T, DINNER), f32),
            pltpu.VMEM((T // TC, BB, TC, DSTATE), f32),
            pltpu.VMEM((T // TC, BB, TC, DSTATE), f32),
        ],
        compiler_params=pltpu.CompilerParams(
            dimension_semantics=("parallel",),
            vmem_limit_bytes=100 * 1024 * 1024,
        ),
    )(fused, ssm_ln_w, ssm_in_proj, cwt, ssm_conv_b, ssm_x_proj,
      ssm_dt_w, ssm_dt_b, alogT, ssm_D, ssm_out_proj)

    logits, emb, s1, s2, s3, s4 = pl.pallas_call(
        _hpc_body,
        out_shape=[
            jax.ShapeDtypeStruct((B, NSIGNS), f32),
            jax.ShapeDtypeStruct((B, DMODEL), f32),
            jax.ShapeDtypeStruct((B, NHS), f32),
            jax.ShapeDtypeStruct((B, NLOC), f32),
            jax.ShapeDtypeStruct((B, NMOV), f32),
            jax.ShapeDtypeStruct((B, NORI), f32),
        ],
        compiler_params=pltpu.CompilerParams(
            vmem_limit_bytes=100 * 1024 * 1024,
        ),
    )(tmean.reshape(B, DMODEL), hpc_w_sign, hpc_b_sign.reshape(1, DMODEL),
      sign_protos, csum.reshape(B, DMODEL), proto_hs, proto_loc,
      proto_mov, proto_ori)
    return (logits, emb, s1, s2, s3, s4)


# speedup vs baseline: 1.1921x; 1.1921x over previous
import functools

import jax
import jax.numpy as jnp
from jax.experimental import pallas as pl
from jax.experimental.pallas import tpu as pltpu

B, T, N, C = 64, 256, 21, 3
HID, HEADS, HDIM, DOUT = 128, 4, 32, 256
CDIM, DMODEL = 64, 256
DINNER, DSTATE, DCONV, DTRANK, L = 512, 16, 4, 16, 4
NSIGNS, NHS, NLOC, NMOV, NORI = 2000, 40, 20, 20, 8
TEMP = 0.07

F = 32
BB = 4
TC = 8


def _agan_body(x_ref, w_in_ref, b_in_ref, wgat_ref, ssrc_ref, sdst_ref,
               w_out_ref, b_out_ref, pdmw_ref, pdmb_ref, wfuse_ref, bfuse_ref,
               fused_ref, csum_ref):
    tb = pl.program_id(1)
    xb = x_ref[...]
    h0 = jnp.maximum(jnp.dot(xb, w_in_ref[...]) + b_in_ref[...], 0.0)
    hf = jnp.dot(h0, wgat_ref[...])
    src4 = jnp.dot(hf, ssrc_ref[...])
    dst4 = jnp.dot(hf, sdst_ref[...])
    srcT = jnp.transpose(src4.reshape(F, N, HEADS), (0, 2, 1)).reshape(
        F, HEADS * N)
    dstrep = jnp.repeat(dst4.reshape(F, N, HEADS), N, axis=2)
    e = srcT[:, None, :] + dstrep
    e = jnp.where(e >= 0.0, e, 0.2 * e)
    m = jnp.max(e, axis=1, keepdims=True)
    ex = jnp.exp(e - m)
    alpha = ex / jnp.sum(ex, axis=1, keepdims=True)
    hf3 = hf.reshape(F, N, HEADS * HDIM)
    aggs = []
    for h in range(HEADS):
        aggs.append(jax.lax.dot_general(
            alpha[:, :, h * N:(h + 1) * N], hf3[:, :, h * HDIM:(h + 1) * HDIM],
            (((1,), (1,)), ((0,), (0,)))))
    agg = jnp.concatenate(aggs, axis=2)
    feat = jnp.maximum(h0.reshape(F, N, HID) + agg, 0.0)
    node = jnp.dot(feat.reshape(F * N, HID), w_out_ref[...]) + b_out_ref[...]
    spatial = jnp.mean(node.reshape(F, N, DOUT), axis=1)
    comps = jnp.maximum(jnp.dot(spatial, pdmw_ref[...]) + pdmb_ref[...], 0.0)
    fused = jnp.dot(comps, wfuse_ref[...]) + bfuse_ref[...]
    fused_ref[...] = fused[None]
    part = jnp.sum(comps, axis=0, keepdims=True)[None]

    @pl.when(tb == 0)
    def _():
        csum_ref[...] = part

    @pl.when(tb > 0)
    def _():
        csum_ref[...] += part


def _ssm_body(fused_ref, lnw_ref, ipr_ref, cwt_ref, cb_ref, xpr_ref,
              dtw_ref, dtb_ref, alogT_ref, d_ref, opr_ref, tmean_ref,
              dt_s, du_s, y_s, bmT_s, cmT_s):
    xx2 = fused_ref[...].reshape(BB * T, DMODEL)
    for l in range(L):
        mu = jnp.mean(xx2, axis=-1, keepdims=True)
        var = jnp.mean((xx2 - mu) ** 2, axis=-1, keepdims=True)
        xn2 = (xx2 - mu) * jax.lax.rsqrt(var + 1e-5) * lnw_ref[l][None, :]
        ysum2 = jnp.zeros((BB * T, DMODEL), jnp.float32)
        for d in range(2):
            fwd = (d == 0)
            xz2 = jnp.dot(xn2, ipr_ref[l, d])
            u3 = xz2[:, :DINNER].reshape(BB, T, DINNER)
            z2 = xz2[:, DINNER:]
            acc = jnp.zeros((BB, T, DINNER), jnp.float32)
            for k in range(DCONV):
                w_k = cwt_ref[l, d, k][None, None, :]
                sh = DCONV - 1 - k
                if sh == 0:
                    term = u3
                elif fwd:
                    term = jnp.concatenate(
                        [jnp.zeros((BB, sh, DINNER), jnp.float32),
                         u3[:, :T - sh, :]], axis=1)
                else:
                    term = jnp.concatenate(
                        [u3[:, sh:, :],
                         jnp.zeros((BB, sh, DINNER), jnp.float32)], axis=1)
                acc = acc + term * w_k
            uc3 = acc + cb_ref[l, d][None, None, :]
            uc3 = uc3 * jax.nn.sigmoid(uc3)
            uc2 = uc3.reshape(BB * T, DINNER)
            xdbl = jnp.dot(uc2, xpr_ref[l, d])
            dt2 = jax.nn.softplus(
                jnp.dot(xdbl[:, :DTRANK], dtw_ref[l, d]) + dtb_ref[l, d][None, :])
            dt_s[...] = dt2.reshape(BB, T, DINNER)
            du_s[...] = (dt2 * uc2).reshape(BB, T, DINNER)
            bmT_s[...] = jnp.transpose(
                xdbl[:, DTRANK:DTRANK + DSTATE].reshape(BB, T // TC, TC, DSTATE),
                (1, 0, 2, 3))
            cmT_s[...] = jnp.transpose(
                xdbl[:, DTRANK + DSTATE:].reshape(BB, T // TC, TC, DSTATE),
                (1, 0, 2, 3))
            at = -jnp.exp(alogT_ref[l, d])

            def chunk(c, h, _fwd=fwd, _at=at):
                t0 = c * TC if _fwd else (T - TC) - c * TC
                ci = c if _fwd else (T // TC - 1) - c
                dtc = dt_s[:, pl.ds(t0, TC), :]
                duc = du_s[:, pl.ds(t0, TC), :]
                bc = bmT_s[ci]
                cc = cmT_s[ci]
                dAc = jnp.exp(dtc[:, :, None, :] * _at[None, None])
                dBc = bc[:, :, :, None] * duc[:, :, None, :]
                hs = [None] * TC
                order = range(TC) if _fwd else range(TC - 1, -1, -1)
                for j in order:
                    h = dAc[:, j] * h + dBc[:, j]
                    hs[j] = h
                hcat = jnp.stack(hs, axis=1)
                y_s[:, pl.ds(t0, TC), :] = jnp.sum(
                    hcat * cc[:, :, :, None], axis=2)
                return h

            h0 = jnp.zeros((BB, DSTATE, DINNER), jnp.float32)
            jax.lax.fori_loop(0, T // TC, chunk, h0)
            y3 = y_s[...] + uc3 * d_ref[l, d][None, None, :]
            g2 = (y3.reshape(BB * T, DINNER)) * (z2 * jax.nn.sigmoid(z2))
            ysum2 = ysum2 + jnp.dot(g2, opr_ref[l, d])
        xx2 = xx2 + ysum2
    tmean_ref[...] = jnp.mean(xx2.reshape(BB, T, DMODEL), axis=1,
                              keepdims=True)


def _cosn(f):
    return f / (jnp.sqrt(jnp.sum(f * f, axis=-1, keepdims=True)) + 1e-8)


def _hpc_body(tm_ref, w_ref, b_ref, sp_ref, cs_ref, phs_ref, plo_ref,
              pmv_ref, por_ref, logits_ref, emb_ref, s1_ref, s2_ref,
              s3_ref, s4_ref):
    emb = jnp.dot(tm_ref[...], w_ref[...]) + b_ref[...]
    emb_ref[...] = emb
    en = _cosn(emb)
    pn = _cosn(sp_ref[...])
    logits_ref[...] = jax.lax.dot_general(
        en, pn, (((1,), (1,)), ((), ()))) * (1.0 / TEMP)
    pooled = cs_ref[...] * (1.0 / T)
    for k, (p_ref, o_ref) in enumerate(((phs_ref, s1_ref), (plo_ref, s2_ref),
                                        (pmv_ref, s3_ref), (por_ref, s4_ref))):
        fn = _cosn(pooled[:, k * CDIM:(k + 1) * CDIM])
        o_ref[...] = jax.lax.dot_general(
            fn, _cosn(p_ref[...]), (((1,), (1,)), ((), ()))) * (1.0 / TEMP)


def _full(spec_shape, ndim=None):
    nd = len(spec_shape)
    return pl.BlockSpec(spec_shape, lambda *_: (0,) * nd)


def kernel(x, agan_w_in, agan_b_in, agan_w_gat, agan_a_src, agan_a_dst,
           agan_w_out, agan_b_out, pdm_w, pdm_b, pdm_w_fuse, pdm_b_fuse,
           ssm_ln_w, ssm_in_proj, ssm_conv_w, ssm_conv_b, ssm_x_proj,
           ssm_dt_w, ssm_dt_b, ssm_A_log, ssm_D, ssm_out_proj,
           hpc_w_sign, hpc_b_sign, sign_protos, proto_hs, proto_loc,
           proto_mov, proto_ori):
    f32 = jnp.float32
    xr = x.reshape(B * T * N, C)
    wgat = agan_w_gat.transpose(1, 0, 2).reshape(HID, HEADS * HDIM)
    headsel = jnp.kron(jnp.eye(HEADS, dtype=f32), jnp.ones((HDIM, 1), f32))
    ssrc = agan_a_src.reshape(-1)[:, None] * headsel
    sdst = agan_a_dst.reshape(-1)[:, None] * headsel
    pdmw = pdm_w.transpose(1, 0, 2).reshape(DOUT, 4 * CDIM)
    pdmb = pdm_b.reshape(1, 4 * CDIM)

    nf = T // F
    fused, csum = pl.pallas_call(
        _agan_body,
        grid=(B, nf),
        in_specs=[
            pl.BlockSpec((F * N, C), lambda b, t: (b * nf + t, 0)),
            _full((C, HID)), _full((1, HID)), _full((HID, HEADS * HDIM)),
            _full((HID, HEADS)), _full((HID, HEADS)),
            _full((HID, DOUT)), _full((1, DOUT)),
            _full((DOUT, 4 * CDIM)), _full((1, 4 * CDIM)),
            _full((4 * CDIM, DMODEL)), _full((1, DMODEL)),
        ],
        out_specs=[
            pl.BlockSpec((1, F, DMODEL), lambda b, t: (b, t, 0)),
            pl.BlockSpec((1, 1, DMODEL), lambda b, t: (b, 0, 0)),
        ],
        out_shape=[
            jax.ShapeDtypeStruct((B, T, DMODEL), f32),
            jax.ShapeDtypeStruct((B, 1, DMODEL), f32),
        ],
        compiler_params=pltpu.CompilerParams(
            dimension_semantics=("parallel", "arbitrary"),
            vmem_limit_bytes=100 * 1024 * 1024,
        ),
    )(xr, agan_w_in, agan_b_in.reshape(1, HID), wgat, ssrc, sdst,
      agan_w_out, agan_b_out.reshape(1, DOUT), pdmw, pdmb,
      pdm_w_fuse, pdm_b_fuse.reshape(1, DMODEL))

    cwt = ssm_conv_w.transpose(0, 1, 3, 2)
    alogT = ssm_A_log.transpose(0, 1, 3, 2)

    tmean = pl.pallas_call(
        _ssm_body,
        grid=(B // BB,),
        in_specs=[
            pl.BlockSpec((BB, T, DMODEL), lambda i: (i, 0, 0)),
            _full((L, DMODEL)),
            _full((L, 2, DMODEL, 2 * DINNER)),
            _full((L, 2, DCONV, DINNER)),
            _full((L, 2, DINNER)),
            _full((L, 2, DINNER, DTRANK + 2 * DSTATE)),
            _full((L, 2, DTRANK, DINNER)),
            _full((L, 2, DINNER)),
            _full((L, 2, DSTATE, DINNER)),
            _full((L, 2, DINNER)),
            _full((L, 2, DINNER, DMODEL)),
        ],
        out_specs=pl.BlockSpec((BB, 1, DMODEL), lambda i: (i, 0, 0)),
        out_shape=jax.ShapeDtypeStruct((B, 1, DMODEL), f32),
        scratch_shapes=[
            pltpu.VMEM((BB, T, DINNER), f32),
            pltpu.VMEM((BB, T, DINNER), f32),
            pltpu.VMEM((BB, T, DINNER), f32),
            pltpu.VMEM((T // TC, BB, TC, DSTATE), f32),
            pltpu.VMEM((T // TC, BB, TC, DSTATE), f32),
        ],
        compiler_params=pltpu.CompilerParams(
            dimension_semantics=("parallel",),
            vmem_limit_bytes=100 * 1024 * 1024,
        ),
    )(fused, ssm_ln_w, ssm_in_proj, cwt, ssm_conv_b, ssm_x_proj,
      ssm_dt_w, ssm_dt_b, alogT, ssm_D, ssm_out_proj)

    logits, emb, s1, s2, s3, s4 = pl.pallas_call(
        _hpc_body,
        out_shape=[
            jax.ShapeDtypeStruct((B, NSIGNS), f32),
            jax.ShapeDtypeStruct((B, DMODEL), f32),
            jax.ShapeDtypeStruct((B, NHS), f32),
            jax.ShapeDtypeStruct((B, NLOC), f32),
            jax.ShapeDtypeStruct((B, NMOV), f32),
            jax.ShapeDtypeStruct((B, NORI), f32),
        ],
        compiler_params=pltpu.CompilerParams(
            vmem_limit_bytes=100 * 1024 * 1024,
        ),
    )(tmean.reshape(B, DMODEL), hpc_w_sign, hpc_b_sign.reshape(1, DMODEL),
      sign_protos, csum.reshape(B, DMODEL), proto_hs, proto_loc,
      proto_mov, proto_ori)
    return (logits, emb, s1, s2, s3, s4)


# speedup vs baseline: 1.3650x; 1.1451x over previous
import functools

import jax
import jax.numpy as jnp
from jax.experimental import pallas as pl
from jax.experimental.pallas import tpu as pltpu

B, T, N, C = 64, 256, 21, 3
HID, HEADS, HDIM, DOUT = 128, 4, 32, 256
CDIM, DMODEL = 64, 256
DINNER, DSTATE, DCONV, DTRANK, L = 512, 16, 4, 16, 4
NSIGNS, NHS, NLOC, NMOV, NORI = 2000, 40, 20, 20, 8
TEMP = 0.07

F = 128
BB = 8
TC = 8


def _agan_body(x_ref, w_in_ref, b_in_ref, wgat_ref, ssrc_ref, sdst_ref,
               w_out_ref, b_out_ref, pdmw_ref, pdmb_ref, wfuse_ref, bfuse_ref,
               fused_ref, csum_ref):
    tb = pl.program_id(1)
    xb = x_ref[...]
    h0 = jnp.maximum(jnp.dot(xb, w_in_ref[...]) + b_in_ref[...], 0.0)
    hf = jnp.dot(h0, wgat_ref[...])
    src4 = jnp.dot(hf, ssrc_ref[...])
    dst4 = jnp.dot(hf, sdst_ref[...])
    srcT = jnp.transpose(src4.reshape(F, N, HEADS), (0, 2, 1)).reshape(
        F, HEADS * N)
    dstrep = jnp.repeat(dst4.reshape(F, N, HEADS), N, axis=2)
    e = srcT[:, None, :] + dstrep
    e = jnp.where(e >= 0.0, e, 0.2 * e)
    m = jnp.max(e, axis=1, keepdims=True)
    ex = jnp.exp(e - m)
    alpha = ex / jnp.sum(ex, axis=1, keepdims=True)
    hf3 = hf.reshape(F, N, HEADS * HDIM)
    aggs = []
    for h in range(HEADS):
        aggs.append(jax.lax.dot_general(
            alpha[:, :, h * N:(h + 1) * N], hf3[:, :, h * HDIM:(h + 1) * HDIM],
            (((1,), (1,)), ((0,), (0,)))))
    agg = jnp.concatenate(aggs, axis=2)
    feat = jnp.maximum(h0.reshape(F, N, HID) + agg, 0.0)
    node = jnp.dot(feat.reshape(F * N, HID), w_out_ref[...]) + b_out_ref[...]
    spatial = jnp.mean(node.reshape(F, N, DOUT), axis=1)
    comps = jnp.maximum(jnp.dot(spatial, pdmw_ref[...]) + pdmb_ref[...], 0.0)
    fused = jnp.dot(comps, wfuse_ref[...]) + bfuse_ref[...]
    fused_ref[...] = fused[None]
    part = jnp.sum(comps, axis=0, keepdims=True)[None]

    @pl.when(tb == 0)
    def _():
        csum_ref[...] = part

    @pl.when(tb > 0)
    def _():
        csum_ref[...] += part


def _ssm_body(fused_ref, lnw_ref, ipr_ref, cwt_ref, cb_ref, xpr_ref,
              dtw_ref, dtb_ref, alogT_ref, d_ref, opr_ref, tmean_ref,
              dt_s, du_s, y_s, bmT_s, cmT_s):
    xx2 = fused_ref[...].reshape(BB * T, DMODEL)
    for l in range(L):
        mu = jnp.mean(xx2, axis=-1, keepdims=True)
        var = jnp.mean((xx2 - mu) ** 2, axis=-1, keepdims=True)
        xn2 = (xx2 - mu) * jax.lax.rsqrt(var + 1e-5) * lnw_ref[l][None, :]
        ysum2 = jnp.zeros((BB * T, DMODEL), jnp.float32)
        for d in range(2):
            fwd = (d == 0)
            xz2 = jnp.dot(xn2, ipr_ref[l, d])
            u3 = xz2[:, :DINNER].reshape(BB, T, DINNER)
            z2 = xz2[:, DINNER:]
            acc = jnp.zeros((BB, T, DINNER), jnp.float32)
            for k in range(DCONV):
                w_k = cwt_ref[l, d, k][None, None, :]
                sh = DCONV - 1 - k
                if sh == 0:
                    term = u3
                elif fwd:
                    term = jnp.concatenate(
                        [jnp.zeros((BB, sh, DINNER), jnp.float32),
                         u3[:, :T - sh, :]], axis=1)
                else:
                    term = jnp.concatenate(
                        [u3[:, sh:, :],
                         jnp.zeros((BB, sh, DINNER), jnp.float32)], axis=1)
                acc = acc + term * w_k
            uc3 = acc + cb_ref[l, d][None, None, :]
            uc3 = uc3 * jax.nn.sigmoid(uc3)
            uc2 = uc3.reshape(BB * T, DINNER)
            xdbl = jnp.dot(uc2, xpr_ref[l, d])
            dt2 = jax.nn.softplus(
                jnp.dot(xdbl[:, :DTRANK], dtw_ref[l, d]) + dtb_ref[l, d][None, :])
            dt_s[...] = dt2.reshape(BB, T, DINNER)
            du_s[...] = (dt2 * uc2).reshape(BB, T, DINNER)
            bmT_s[...] = jnp.transpose(
                xdbl[:, DTRANK:DTRANK + DSTATE].reshape(BB, T // TC, TC, DSTATE),
                (1, 0, 2, 3))
            cmT_s[...] = jnp.transpose(
                xdbl[:, DTRANK + DSTATE:].reshape(BB, T // TC, TC, DSTATE),
                (1, 0, 2, 3))
            at = -jnp.exp(alogT_ref[l, d])

            def chunk(c, h, _fwd=fwd, _at=at):
                t0 = c * TC if _fwd else (T - TC) - c * TC
                ci = c if _fwd else (T // TC - 1) - c
                dtc = dt_s[:, pl.ds(t0, TC), :]
                duc = du_s[:, pl.ds(t0, TC), :]
                bc = bmT_s[ci]
                cc = cmT_s[ci]
                dAc = jnp.exp(dtc[:, :, None, :] * _at[None, None])
                dBc = bc[:, :, :, None] * duc[:, :, None, :]
                hs = [None] * TC
                order = range(TC) if _fwd else range(TC - 1, -1, -1)
                for j in order:
                    h = dAc[:, j] * h + dBc[:, j]
                    hs[j] = h
                hcat = jnp.stack(hs, axis=1)
                y_s[:, pl.ds(t0, TC), :] = jnp.sum(
                    hcat * cc[:, :, :, None], axis=2)
                return h

            h0 = jnp.zeros((BB, DSTATE, DINNER), jnp.float32)
            jax.lax.fori_loop(0, T // TC, chunk, h0)
            y3 = y_s[...] + uc3 * d_ref[l, d][None, None, :]
            g2 = (y3.reshape(BB * T, DINNER)) * (z2 * jax.nn.sigmoid(z2))
            ysum2 = ysum2 + jnp.dot(g2, opr_ref[l, d])
        xx2 = xx2 + ysum2
    tmean_ref[...] = jnp.mean(xx2.reshape(BB, T, DMODEL), axis=1,
                              keepdims=True)


def _cosn(f):
    return f / (jnp.sqrt(jnp.sum(f * f, axis=-1, keepdims=True)) + 1e-8)


def _hpc_body(tm_ref, w_ref, b_ref, sp_ref, cs_ref, phs_ref, plo_ref,
              pmv_ref, por_ref, logits_ref, emb_ref, s1_ref, s2_ref,
              s3_ref, s4_ref):
    emb = jnp.dot(tm_ref[...], w_ref[...]) + b_ref[...]
    emb_ref[...] = emb
    en = _cosn(emb)
    pn = _cosn(sp_ref[...])
    logits_ref[...] = jax.lax.dot_general(
        en, pn, (((1,), (1,)), ((), ()))) * (1.0 / TEMP)
    pooled = cs_ref[...] * (1.0 / T)
    for k, (p_ref, o_ref) in enumerate(((phs_ref, s1_ref), (plo_ref, s2_ref),
                                        (pmv_ref, s3_ref), (por_ref, s4_ref))):
        fn = _cosn(pooled[:, k * CDIM:(k + 1) * CDIM])
        o_ref[...] = jax.lax.dot_general(
            fn, _cosn(p_ref[...]), (((1,), (1,)), ((), ()))) * (1.0 / TEMP)


def _full(spec_shape, ndim=None):
    nd = len(spec_shape)
    return pl.BlockSpec(spec_shape, lambda *_: (0,) * nd)


def kernel(x, agan_w_in, agan_b_in, agan_w_gat, agan_a_src, agan_a_dst,
           agan_w_out, agan_b_out, pdm_w, pdm_b, pdm_w_fuse, pdm_b_fuse,
           ssm_ln_w, ssm_in_proj, ssm_conv_w, ssm_conv_b, ssm_x_proj,
           ssm_dt_w, ssm_dt_b, ssm_A_log, ssm_D, ssm_out_proj,
           hpc_w_sign, hpc_b_sign, sign_protos, proto_hs, proto_loc,
           proto_mov, proto_ori):
    f32 = jnp.float32
    xr = x.reshape(B * T * N, C)
    wgat = agan_w_gat.transpose(1, 0, 2).reshape(HID, HEADS * HDIM)
    headsel = jnp.kron(jnp.eye(HEADS, dtype=f32), jnp.ones((HDIM, 1), f32))
    ssrc = agan_a_src.reshape(-1)[:, None] * headsel
    sdst = agan_a_dst.reshape(-1)[:, None] * headsel
    pdmw = pdm_w.transpose(1, 0, 2).reshape(DOUT, 4 * CDIM)
    pdmb = pdm_b.reshape(1, 4 * CDIM)

    nf = T // F
    fused, csum = pl.pallas_call(
        _agan_body,
        grid=(B, nf),
        in_specs=[
            pl.BlockSpec((F * N, C), lambda b, t: (b * nf + t, 0)),
            _full((C, HID)), _full((1, HID)), _full((HID, HEADS * HDIM)),
            _full((HID, HEADS)), _full((HID, HEADS)),
            _full((HID, DOUT)), _full((1, DOUT)),
            _full((DOUT, 4 * CDIM)), _full((1, 4 * CDIM)),
            _full((4 * CDIM, DMODEL)), _full((1, DMODEL)),
        ],
        out_specs=[
            pl.BlockSpec((1, F, DMODEL), lambda b, t: (b, t, 0)),
            pl.BlockSpec((1, 1, DMODEL), lambda b, t: (b, 0, 0)),
        ],
        out_shape=[
            jax.ShapeDtypeStruct((B, T, DMODEL), f32),
            jax.ShapeDtypeStruct((B, 1, DMODEL), f32),
        ],
        compiler_params=pltpu.CompilerParams(
            dimension_semantics=("parallel", "arbitrary"),
            vmem_limit_bytes=100 * 1024 * 1024,
        ),
    )(xr, agan_w_in, agan_b_in.reshape(1, HID), wgat, ssrc, sdst,
      agan_w_out, agan_b_out.reshape(1, DOUT), pdmw, pdmb,
      pdm_w_fuse, pdm_b_fuse.reshape(1, DMODEL))

    cwt = ssm_conv_w.transpose(0, 1, 3, 2)
    alogT = ssm_A_log.transpose(0, 1, 3, 2)

    tmean = pl.pallas_call(
        _ssm_body,
        grid=(B // BB,),
        in_specs=[
            pl.BlockSpec((BB, T, DMODEL), lambda i: (i, 0, 0)),
            _full((L, DMODEL)),
            _full((L, 2, DMODEL, 2 * DINNER)),
            _full((L, 2, DCONV, DINNER)),
            _full((L, 2, DINNER)),
            _full((L, 2, DINNER, DTRANK + 2 * DSTATE)),
            _full((L, 2, DTRANK, DINNER)),
            _full((L, 2, DINNER)),
            _full((L, 2, DSTATE, DINNER)),
            _full((L, 2, DINNER)),
            _full((L, 2, DINNER, DMODEL)),
        ],
        out_specs=pl.BlockSpec((BB, 1, DMODEL), lambda i: (i, 0, 0)),
        out_shape=jax.ShapeDtypeStruct((B, 1, DMODEL), f32),
        scratch_shapes=[
            pltpu.VMEM((BB, T, DINNER), f32),
            pltpu.VMEM((BB, T, DINNER), f32),
            pltpu.VMEM((BB, T, DINNER), f32),
            pltpu.VMEM((T // TC, BB, TC, DSTATE), f32),
            pltpu.VMEM((T // TC, BB, TC, DSTATE), f32),
        ],
        compiler_params=pltpu.CompilerParams(
            dimension_semantics=("parallel",),
            vmem_limit_bytes=100 * 1024 * 1024,
        ),
    )(fused, ssm_ln_w, ssm_in_proj, cwt, ssm_conv_b, ssm_x_proj,
      ssm_dt_w, ssm_dt_b, alogT, ssm_D, ssm_out_proj)

    logits, emb, s1, s2, s3, s4 = pl.pallas_call(
        _hpc_body,
        out_shape=[
            jax.ShapeDtypeStruct((B, NSIGNS), f32),
            jax.ShapeDtypeStruct((B, DMODEL), f32),
            jax.ShapeDtypeStruct((B, NHS), f32),
            jax.ShapeDtypeStruct((B, NLOC), f32),
            jax.ShapeDtypeStruct((B, NMOV), f32),
            jax.ShapeDtypeStruct((B, NORI), f32),
        ],
        compiler_params=pltpu.CompilerParams(
            vmem_limit_bytes=100 * 1024 * 1024,
        ),
    )(tmean.reshape(B, DMODEL), hpc_w_sign, hpc_b_sign.reshape(1, DMODEL),
      sign_protos, csum.reshape(B, DMODEL), proto_hs, proto_loc,
      proto_mov, proto_ori)
    return (logits, emb, s1, s2, s3, s4)


# speedup vs baseline: 1.6261x; 1.1913x over previous
import functools

import jax
import jax.numpy as jnp
from jax.experimental import pallas as pl
from jax.experimental.pallas import tpu as pltpu

B, T, N, C = 64, 256, 21, 3
HID, HEADS, HDIM, DOUT = 128, 4, 32, 256
CDIM, DMODEL = 64, 256
DINNER, DSTATE, DCONV, DTRANK, L = 512, 16, 4, 16, 4
NSIGNS, NHS, NLOC, NMOV, NORI = 2000, 40, 20, 20, 8
TEMP = 0.07

F = 128
NP = 24
BB = 8
TC = 8


def _agan_body(x_ref, w_in_ref, b_in_ref, wgat_ref, ssrc_ref, sdst_ref,
               rep_ref, w_out_ref, b_out_ref, pdmw_ref, pdmb_ref, wfuse_ref,
               bfuse_ref, fused_ref, csum_ref):
    tb = pl.program_id(1)
    xb = x_ref[...]
    h0 = jnp.maximum(jnp.dot(xb, w_in_ref[...]) + b_in_ref[...], 0.0)
    hf = jnp.dot(h0, wgat_ref[...])
    src4 = jnp.dot(hf, ssrc_ref[...])
    srcT = jnp.transpose(src4.reshape(F, NP, HEADS), (0, 2, 1)).reshape(
        F, HEADS * NP)
    dstrep = jnp.dot(hf, jnp.dot(sdst_ref[...], rep_ref[...])).reshape(
        F, NP, HEADS * NP)
    e = srcT[:, None, :] + dstrep
    e = jnp.where(e >= 0.0, e, 0.2 * e)
    jmask = jax.lax.broadcasted_iota(jnp.int32, (F, NP, 1), 1) < N
    e = jnp.where(jmask, e, -1e30)
    m = jnp.max(e, axis=1, keepdims=True)
    ex = jnp.exp(e - m)
    alpha = ex / jnp.sum(ex, axis=1, keepdims=True)
    hf3 = hf.reshape(F, NP, HEADS * HDIM)
    aggs = []
    for h in range(HEADS):
        aggs.append(jax.lax.dot_general(
            alpha[:, :, h * NP:(h + 1) * NP], hf3[:, :, h * HDIM:(h + 1) * HDIM],
            (((1,), (1,)), ((0,), (0,)))))
    agg = jnp.concatenate(aggs, axis=2)
    feat = jnp.maximum(h0.reshape(F, NP, HID) + agg, 0.0)
    node = jnp.dot(feat.reshape(F * NP, HID), w_out_ref[...]) + b_out_ref[...]
    node3 = node.reshape(F, NP, DOUT)
    imask = jax.lax.broadcasted_iota(jnp.int32, (F, NP, 1), 1) < N
    spatial = jnp.sum(jnp.where(imask, node3, 0.0), axis=1) * (1.0 / N)
    comps = jnp.maximum(jnp.dot(spatial, pdmw_ref[...]) + pdmb_ref[...], 0.0)
    fused = jnp.dot(comps, wfuse_ref[...]) + bfuse_ref[...]
    fused_ref[...] = fused[None]
    part = jnp.sum(comps, axis=0, keepdims=True)[None]

    @pl.when(tb == 0)
    def _():
        csum_ref[...] = part

    @pl.when(tb > 0)
    def _():
        csum_ref[...] += part


def _ssm_body(fused_ref, lnw_ref, ipr_ref, cwt_ref, cb_ref, xpr_ref,
              dtw_ref, dtb_ref, alogT_ref, d_ref, opr_ref, tmean_ref,
              dt_s, du_s, y_s, bmT_s, cmT_s):
    xx2 = fused_ref[...].reshape(BB * T, DMODEL)
    for l in range(L):
        mu = jnp.mean(xx2, axis=-1, keepdims=True)
        var = jnp.mean((xx2 - mu) ** 2, axis=-1, keepdims=True)
        xn2 = (xx2 - mu) * jax.lax.rsqrt(var + 1e-5) * lnw_ref[l][None, :]
        ysum2 = jnp.zeros((BB * T, DMODEL), jnp.float32)
        for d in range(2):
            fwd = (d == 0)
            xz2 = jnp.dot(xn2, ipr_ref[l, d])
            u3 = xz2[:, :DINNER].reshape(BB, T, DINNER)
            z2 = xz2[:, DINNER:]
            acc = jnp.zeros((BB, T, DINNER), jnp.float32)
            for k in range(DCONV):
                w_k = cwt_ref[l, d, k][None, None, :]
                sh = DCONV - 1 - k
                if sh == 0:
                    term = u3
                elif fwd:
                    term = jnp.concatenate(
                        [jnp.zeros((BB, sh, DINNER), jnp.float32),
                         u3[:, :T - sh, :]], axis=1)
                else:
                    term = jnp.concatenate(
                        [u3[:, sh:, :],
                         jnp.zeros((BB, sh, DINNER), jnp.float32)], axis=1)
                acc = acc + term * w_k
            uc3 = acc + cb_ref[l, d][None, None, :]
            uc3 = uc3 * jax.nn.sigmoid(uc3)
            uc2 = uc3.reshape(BB * T, DINNER)
            xdbl = jnp.dot(uc2, xpr_ref[l, d])
            dt2 = jax.nn.softplus(
                jnp.dot(xdbl[:, :DTRANK], dtw_ref[l, d]) + dtb_ref[l, d][None, :])
            dt_s[...] = dt2.reshape(BB, T, DINNER)
            du_s[...] = (dt2 * uc2).reshape(BB, T, DINNER)
            bmT_s[...] = jnp.transpose(
                xdbl[:, DTRANK:DTRANK + DSTATE].reshape(BB, T // TC, TC, DSTATE),
                (1, 0, 2, 3))
            cmT_s[...] = jnp.transpose(
                xdbl[:, DTRANK + DSTATE:].reshape(BB, T // TC, TC, DSTATE),
                (1, 0, 2, 3))
            at = -jnp.exp(alogT_ref[l, d])

            def chunk(c, h, _fwd=fwd, _at=at):
                t0 = c * TC if _fwd else (T - TC) - c * TC
                ci = c if _fwd else (T // TC - 1) - c
                dtc = dt_s[:, pl.ds(t0, TC), :]
                duc = du_s[:, pl.ds(t0, TC), :]
                bc = bmT_s[ci]
                cc = cmT_s[ci]
                dAc = jnp.exp(dtc[:, :, None, :] * _at[None, None])
                dBc = bc[:, :, :, None] * duc[:, :, None, :]
                hs = [None] * TC
                order = range(TC) if _fwd else range(TC - 1, -1, -1)
                for j in order:
                    h = dAc[:, j] * h + dBc[:, j]
                    hs[j] = h
                hcat = jnp.stack(hs, axis=1)
                y_s[:, pl.ds(t0, TC), :] = jnp.sum(
                    hcat * cc[:, :, :, None], axis=2)
                return h

            h0 = jnp.zeros((BB, DSTATE, DINNER), jnp.float32)
            jax.lax.fori_loop(0, T // TC, chunk, h0)
            y3 = y_s[...] + uc3 * d_ref[l, d][None, None, :]
            g2 = (y3.reshape(BB * T, DINNER)) * (z2 * jax.nn.sigmoid(z2))
            ysum2 = ysum2 + jnp.dot(g2, opr_ref[l, d])
        xx2 = xx2 + ysum2
    tmean_ref[...] = jnp.mean(xx2.reshape(BB, T, DMODEL), axis=1,
                              keepdims=True)


def _cosn(f):
    return f / (jnp.sqrt(jnp.sum(f * f, axis=-1, keepdims=True)) + 1e-8)


def _hpc_body(tm_ref, w_ref, b_ref, sp_ref, cs_ref, phs_ref, plo_ref,
              pmv_ref, por_ref, logits_ref, emb_ref, s1_ref, s2_ref,
              s3_ref, s4_ref):
    emb = jnp.dot(tm_ref[...], w_ref[...]) + b_ref[...]
    emb_ref[...] = emb
    en = _cosn(emb)
    pn = _cosn(sp_ref[...])
    logits_ref[...] = jax.lax.dot_general(
        en, pn, (((1,), (1,)), ((), ()))) * (1.0 / TEMP)
    pooled = cs_ref[...] * (1.0 / T)
    for k, (p_ref, o_ref) in enumerate(((phs_ref, s1_ref), (plo_ref, s2_ref),
                                        (pmv_ref, s3_ref), (por_ref, s4_ref))):
        fn = _cosn(pooled[:, k * CDIM:(k + 1) * CDIM])
        o_ref[...] = jax.lax.dot_general(
            fn, _cosn(p_ref[...]), (((1,), (1,)), ((), ()))) * (1.0 / TEMP)


def _full(spec_shape, ndim=None):
    nd = len(spec_shape)
    return pl.BlockSpec(spec_shape, lambda *_: (0,) * nd)


def kernel(x, agan_w_in, agan_b_in, agan_w_gat, agan_a_src, agan_a_dst,
           agan_w_out, agan_b_out, pdm_w, pdm_b, pdm_w_fuse, pdm_b_fuse,
           ssm_ln_w, ssm_in_proj, ssm_conv_w, ssm_conv_b, ssm_x_proj,
           ssm_dt_w, ssm_dt_b, ssm_A_log, ssm_D, ssm_out_proj,
           hpc_w_sign, hpc_b_sign, sign_protos, proto_hs, proto_loc,
           proto_mov, proto_ori):
    f32 = jnp.float32
    xr = jnp.pad(x, ((0, 0), (0, 0), (0, NP - N), (0, 0))).reshape(
        B * T * NP, C)
    rep = jnp.kron(jnp.eye(HEADS, dtype=f32), jnp.ones((1, NP), f32))
    wgat = agan_w_gat.transpose(1, 0, 2).reshape(HID, HEADS * HDIM)
    headsel = jnp.kron(jnp.eye(HEADS, dtype=f32), jnp.ones((HDIM, 1), f32))
    ssrc = agan_a_src.reshape(-1)[:, None] * headsel
    sdst = agan_a_dst.reshape(-1)[:, None] * headsel
    pdmw = pdm_w.transpose(1, 0, 2).reshape(DOUT, 4 * CDIM)
    pdmb = pdm_b.reshape(1, 4 * CDIM)

    nf = T // F
    fused, csum = pl.pallas_call(
        _agan_body,
        grid=(B, nf),
        in_specs=[
            pl.BlockSpec((F * NP, C), lambda b, t: (b * nf + t, 0)),
            _full((C, HID)), _full((1, HID)), _full((HID, HEADS * HDIM)),
            _full((HID, HEADS)), _full((HID, HEADS)),
            _full((HEADS, HEADS * NP)),
            _full((HID, DOUT)), _full((1, DOUT)),
            _full((DOUT, 4 * CDIM)), _full((1, 4 * CDIM)),
            _full((4 * CDIM, DMODEL)), _full((1, DMODEL)),
        ],
        out_specs=[
            pl.BlockSpec((1, F, DMODEL), lambda b, t: (b, t, 0)),
            pl.BlockSpec((1, 1, DMODEL), lambda b, t: (b, 0, 0)),
        ],
        out_shape=[
            jax.ShapeDtypeStruct((B, T, DMODEL), f32),
            jax.ShapeDtypeStruct((B, 1, DMODEL), f32),
        ],
        compiler_params=pltpu.CompilerParams(
            dimension_semantics=("parallel", "arbitrary"),
            vmem_limit_bytes=100 * 1024 * 1024,
        ),
    )(xr, agan_w_in, agan_b_in.reshape(1, HID), wgat, ssrc, sdst, rep,
      agan_w_out, agan_b_out.reshape(1, DOUT), pdmw, pdmb,
      pdm_w_fuse, pdm_b_fuse.reshape(1, DMODEL))

    cwt = ssm_conv_w.transpose(0, 1, 3, 2)
    alogT = ssm_A_log.transpose(0, 1, 3, 2)

    tmean = pl.pallas_call(
        _ssm_body,
        grid=(B // BB,),
        in_specs=[
            pl.BlockSpec((BB, T, DMODEL), lambda i: (i, 0, 0)),
            _full((L, DMODEL)),
            _full((L, 2, DMODEL, 2 * DINNER)),
            _full((L, 2, DCONV, DINNER)),
            _full((L, 2, DINNER)),
            _full((L, 2, DINNER, DTRANK + 2 * DSTATE)),
            _full((L, 2, DTRANK, DINNER)),
            _full((L, 2, DINNER)),
            _full((L, 2, DSTATE, DINNER)),
            _full((L, 2, DINNER)),
            _full((L, 2, DINNER, DMODEL)),
        ],
        out_specs=pl.BlockSpec((BB, 1, DMODEL), lambda i: (i, 0, 0)),
        out_shape=jax.ShapeDtypeStruct((B, 1, DMODEL), f32),
        scratch_shapes=[
            pltpu.VMEM((BB, T, DINNER), f32),
            pltpu.VMEM((BB, T, DINNER), f32),
            pltpu.VMEM((BB, T, DINNER), f32),
            pltpu.VMEM((T // TC, BB, TC, DSTATE), f32),
            pltpu.VMEM((T // TC, BB, TC, DSTATE), f32),
        ],
        compiler_params=pltpu.CompilerParams(
            dimension_semantics=("parallel",),
            vmem_limit_bytes=100 * 1024 * 1024,
        ),
    )(fused, ssm_ln_w, ssm_in_proj, cwt, ssm_conv_b, ssm_x_proj,
      ssm_dt_w, ssm_dt_b, alogT, ssm_D, ssm_out_proj)

    logits, emb, s1, s2, s3, s4 = pl.pallas_call(
        _hpc_body,
        out_shape=[
            jax.ShapeDtypeStruct((B, NSIGNS), f32),
            jax.ShapeDtypeStruct((B, DMODEL), f32),
            jax.ShapeDtypeStruct((B, NHS), f32),
            jax.ShapeDtypeStruct((B, NLOC), f32),
            jax.ShapeDtypeStruct((B, NMOV), f32),
            jax.ShapeDtypeStruct((B, NORI), f32),
        ],
        compiler_params=pltpu.CompilerParams(
            vmem_limit_bytes=100 * 1024 * 1024,
        ),
    )(tmean.reshape(B, DMODEL), hpc_w_sign, hpc_b_sign.reshape(1, DMODEL),
      sign_protos, csum.reshape(B, DMODEL), proto_hs, proto_loc,
      proto_mov, proto_ori)
    return (logits, emb, s1, s2, s3, s4)


# speedup vs baseline: 1.6397x; 1.0084x over previous
import functools

import jax
import jax.numpy as jnp
from jax.experimental import pallas as pl
from jax.experimental.pallas import tpu as pltpu

B, T, N, C = 64, 256, 21, 3
HID, HEADS, HDIM, DOUT = 128, 4, 32, 256
CDIM, DMODEL = 64, 256
DINNER, DSTATE, DCONV, DTRANK, L = 512, 16, 4, 16, 4
NSIGNS, NHS, NLOC, NMOV, NORI = 2000, 40, 20, 20, 8
TEMP = 0.07

F = 256
NP = 24
BB = 8
TC = 16


def _agan_body(x_ref, w_in_ref, b_in_ref, wgat_ref, ssrc_ref, sdst_ref,
               rep_ref, w_out_ref, b_out_ref, pdmw_ref, pdmb_ref, wfuse_ref,
               bfuse_ref, fused_ref, csum_ref):
    tb = pl.program_id(1)
    xb = x_ref[...]
    h0 = jnp.maximum(jnp.dot(xb, w_in_ref[...]) + b_in_ref[...], 0.0)
    hf = jnp.dot(h0, wgat_ref[...])
    src4 = jnp.dot(hf, ssrc_ref[...])
    srcT = jnp.transpose(src4.reshape(F, NP, HEADS), (0, 2, 1)).reshape(
        F, HEADS * NP)
    dstrep = jnp.dot(hf, jnp.dot(sdst_ref[...], rep_ref[...])).reshape(
        F, NP, HEADS * NP)
    e = srcT[:, None, :] + dstrep
    e = jnp.where(e >= 0.0, e, 0.2 * e)
    jmask = jax.lax.broadcasted_iota(jnp.int32, (F, NP, 1), 1) < N
    e = jnp.where(jmask, e, -1e30)
    m = jnp.max(e, axis=1, keepdims=True)
    ex = jnp.exp(e - m)
    alpha = ex / jnp.sum(ex, axis=1, keepdims=True)
    hf3 = hf.reshape(F, NP, HEADS * HDIM)
    aggs = []
    for h in range(HEADS):
        aggs.append(jax.lax.dot_general(
            alpha[:, :, h * NP:(h + 1) * NP], hf3[:, :, h * HDIM:(h + 1) * HDIM],
            (((1,), (1,)), ((0,), (0,)))))
    agg = jnp.concatenate(aggs, axis=2)
    feat = jnp.maximum(h0.reshape(F, NP, HID) + agg, 0.0)
    node = jnp.dot(feat.reshape(F * NP, HID), w_out_ref[...]) + b_out_ref[...]
    node3 = node.reshape(F, NP, DOUT)
    imask = jax.lax.broadcasted_iota(jnp.int32, (F, NP, 1), 1) < N
    spatial = jnp.sum(jnp.where(imask, node3, 0.0), axis=1) * (1.0 / N)
    comps = jnp.maximum(jnp.dot(spatial, pdmw_ref[...]) + pdmb_ref[...], 0.0)
    fused = jnp.dot(comps, wfuse_ref[...]) + bfuse_ref[...]
    fused_ref[...] = fused[None]
    part = jnp.sum(comps, axis=0, keepdims=True)[None]

    @pl.when(tb == 0)
    def _():
        csum_ref[...] = part

    @pl.when(tb > 0)
    def _():
        csum_ref[...] += part


def _ssm_body(fused_ref, lnw_ref, ipr_ref, cwt_ref, cb_ref, xpr_ref,
              dtw_ref, dtb_ref, alogT_ref, d_ref, opr_ref, tmean_ref,
              dt_s, du_s, y_s, bmT_s, cmT_s):
    xx2 = fused_ref[...].reshape(BB * T, DMODEL)
    for l in range(L):
        mu = jnp.mean(xx2, axis=-1, keepdims=True)
        var = jnp.mean((xx2 - mu) ** 2, axis=-1, keepdims=True)
        xn2 = (xx2 - mu) * jax.lax.rsqrt(var + 1e-5) * lnw_ref[l][None, :]
        ysum2 = jnp.zeros((BB * T, DMODEL), jnp.float32)
        for d in range(2):
            fwd = (d == 0)
            xz2 = jnp.dot(xn2, ipr_ref[l, d])
            u3 = xz2[:, :DINNER].reshape(BB, T, DINNER)
            z2 = xz2[:, DINNER:]
            acc = jnp.zeros((BB, T, DINNER), jnp.float32)
            for k in range(DCONV):
                w_k = cwt_ref[l, d, k][None, None, :]
                sh = DCONV - 1 - k
                if sh == 0:
                    term = u3
                elif fwd:
                    term = jnp.concatenate(
                        [jnp.zeros((BB, sh, DINNER), jnp.float32),
                         u3[:, :T - sh, :]], axis=1)
                else:
                    term = jnp.concatenate(
                        [u3[:, sh:, :],
                         jnp.zeros((BB, sh, DINNER), jnp.float32)], axis=1)
                acc = acc + term * w_k
            uc3 = acc + cb_ref[l, d][None, None, :]
            uc3 = uc3 * jax.nn.sigmoid(uc3)
            uc2 = uc3.reshape(BB * T, DINNER)
            xdbl = jnp.dot(uc2, xpr_ref[l, d])
            dt2 = jax.nn.softplus(
                jnp.dot(xdbl[:, :DTRANK], dtw_ref[l, d]) + dtb_ref[l, d][None, :])
            dt_s[...] = dt2.reshape(BB, T, DINNER)
            du_s[...] = (dt2 * uc2).reshape(BB, T, DINNER)
            bmT_s[...] = jnp.transpose(
                xdbl[:, DTRANK:DTRANK + DSTATE].reshape(BB, T // TC, TC, DSTATE),
                (1, 0, 2, 3))
            cmT_s[...] = jnp.transpose(
                xdbl[:, DTRANK + DSTATE:].reshape(BB, T // TC, TC, DSTATE),
                (1, 0, 2, 3))
            at = -jnp.exp(alogT_ref[l, d])

            def chunk(c, h, _fwd=fwd, _at=at):
                t0 = c * TC if _fwd else (T - TC) - c * TC
                ci = c if _fwd else (T // TC - 1) - c
                dtc = dt_s[:, pl.ds(t0, TC), :]
                duc = du_s[:, pl.ds(t0, TC), :]
                bc = bmT_s[ci]
                cc = cmT_s[ci]
                dAc = jnp.exp(dtc[:, :, None, :] * _at[None, None])
                dBc = bc[:, :, :, None] * duc[:, :, None, :]
                hs = [None] * TC
                order = range(TC) if _fwd else range(TC - 1, -1, -1)
                for j in order:
                    h = dAc[:, j] * h + dBc[:, j]
                    hs[j] = h
                hcat = jnp.stack(hs, axis=1)
                y_s[:, pl.ds(t0, TC), :] = jnp.sum(
                    hcat * cc[:, :, :, None], axis=2)
                return h

            h0 = jnp.zeros((BB, DSTATE, DINNER), jnp.float32)
            jax.lax.fori_loop(0, T // TC, chunk, h0)
            y3 = y_s[...] + uc3 * d_ref[l, d][None, None, :]
            g2 = (y3.reshape(BB * T, DINNER)) * (z2 * jax.nn.sigmoid(z2))
            ysum2 = ysum2 + jnp.dot(g2, opr_ref[l, d])
        xx2 = xx2 + ysum2
    tmean_ref[...] = jnp.mean(xx2.reshape(BB, T, DMODEL), axis=1,
                              keepdims=True)


def _cosn(f):
    return f / (jnp.sqrt(jnp.sum(f * f, axis=-1, keepdims=True)) + 1e-8)


def _hpc_body(tm_ref, w_ref, b_ref, sp_ref, cs_ref, phs_ref, plo_ref,
              pmv_ref, por_ref, logits_ref, emb_ref, s1_ref, s2_ref,
              s3_ref, s4_ref):
    emb = jnp.dot(tm_ref[...], w_ref[...]) + b_ref[...]
    emb_ref[...] = emb
    en = _cosn(emb)
    pn = _cosn(sp_ref[...])
    logits_ref[...] = jax.lax.dot_general(
        en, pn, (((1,), (1,)), ((), ()))) * (1.0 / TEMP)
    pooled = cs_ref[...] * (1.0 / T)
    for k, (p_ref, o_ref) in enumerate(((phs_ref, s1_ref), (plo_ref, s2_ref),
                                        (pmv_ref, s3_ref), (por_ref, s4_ref))):
        fn = _cosn(pooled[:, k * CDIM:(k + 1) * CDIM])
        o_ref[...] = jax.lax.dot_general(
            fn, _cosn(p_ref[...]), (((1,), (1,)), ((), ()))) * (1.0 / TEMP)


def _full(spec_shape, ndim=None):
    nd = len(spec_shape)
    return pl.BlockSpec(spec_shape, lambda *_: (0,) * nd)


def kernel(x, agan_w_in, agan_b_in, agan_w_gat, agan_a_src, agan_a_dst,
           agan_w_out, agan_b_out, pdm_w, pdm_b, pdm_w_fuse, pdm_b_fuse,
           ssm_ln_w, ssm_in_proj, ssm_conv_w, ssm_conv_b, ssm_x_proj,
           ssm_dt_w, ssm_dt_b, ssm_A_log, ssm_D, ssm_out_proj,
           hpc_w_sign, hpc_b_sign, sign_protos, proto_hs, proto_loc,
           proto_mov, proto_ori):
    f32 = jnp.float32
    xr = jnp.pad(x, ((0, 0), (0, 0), (0, NP - N), (0, 0))).reshape(
        B * T * NP, C)
    rep = jnp.kron(jnp.eye(HEADS, dtype=f32), jnp.ones((1, NP), f32))
    wgat = agan_w_gat.transpose(1, 0, 2).reshape(HID, HEADS * HDIM)
    headsel = jnp.kron(jnp.eye(HEADS, dtype=f32), jnp.ones((HDIM, 1), f32))
    ssrc = agan_a_src.reshape(-1)[:, None] * headsel
    sdst = agan_a_dst.reshape(-1)[:, None] * headsel
    pdmw = pdm_w.transpose(1, 0, 2).reshape(DOUT, 4 * CDIM)
    pdmb = pdm_b.reshape(1, 4 * CDIM)

    nf = T // F
    fused, csum = pl.pallas_call(
        _agan_body,
        grid=(B, nf),
        in_specs=[
            pl.BlockSpec((F * NP, C), lambda b, t: (b * nf + t, 0)),
            _full((C, HID)), _full((1, HID)), _full((HID, HEADS * HDIM)),
            _full((HID, HEADS)), _full((HID, HEADS)),
            _full((HEADS, HEADS * NP)),
            _full((HID, DOUT)), _full((1, DOUT)),
            _full((DOUT, 4 * CDIM)), _full((1, 4 * CDIM)),
            _full((4 * CDIM, DMODEL)), _full((1, DMODEL)),
        ],
        out_specs=[
            pl.BlockSpec((1, F, DMODEL), lambda b, t: (b, t, 0)),
            pl.BlockSpec((1, 1, DMODEL), lambda b, t: (b, 0, 0)),
        ],
        out_shape=[
            jax.ShapeDtypeStruct((B, T, DMODEL), f32),
            jax.ShapeDtypeStruct((B, 1, DMODEL), f32),
        ],
        compiler_params=pltpu.CompilerParams(
            dimension_semantics=("parallel", "arbitrary"),
            vmem_limit_bytes=100 * 1024 * 1024,
        ),
    )(xr, agan_w_in, agan_b_in.reshape(1, HID), wgat, ssrc, sdst, rep,
      agan_w_out, agan_b_out.reshape(1, DOUT), pdmw, pdmb,
      pdm_w_fuse, pdm_b_fuse.reshape(1, DMODEL))

    cwt = ssm_conv_w.transpose(0, 1, 3, 2)
    alogT = ssm_A_log.transpose(0, 1, 3, 2)

    tmean = pl.pallas_call(
        _ssm_body,
        grid=(B // BB,),
        in_specs=[
            pl.BlockSpec((BB, T, DMODEL), lambda i: (i, 0, 0)),
            _full((L, DMODEL)),
            _full((L, 2, DMODEL, 2 * DINNER)),
            _full((L, 2, DCONV, DINNER)),
            _full((L, 2, DINNER)),
            _full((L, 2, DINNER, DTRANK + 2 * DSTATE)),
            _full((L, 2, DTRANK, DINNER)),
            _full((L, 2, DINNER)),
            _full((L, 2, DSTATE, DINNER)),
            _full((L, 2, DINNER)),
            _full((L, 2, DINNER, DMODEL)),
        ],
        out_specs=pl.BlockSpec((BB, 1, DMODEL), lambda i: (i, 0, 0)),
        out_shape=jax.ShapeDtypeStruct((B, 1, DMODEL), f32),
        scratch_shapes=[
            pltpu.VMEM((BB, T, DINNER), f32),
            pltpu.VMEM((BB, T, DINNER), f32),
            pltpu.VMEM((BB, T, DINNER), f32),
            pltpu.VMEM((T // TC, BB, TC, DSTATE), f32),
            pltpu.VMEM((T // TC, BB, TC, DSTATE), f32),
        ],
        compiler_params=pltpu.CompilerParams(
            dimension_semantics=("parallel",),
            vmem_limit_bytes=100 * 1024 * 1024,
        ),
    )(fused, ssm_ln_w, ssm_in_proj, cwt, ssm_conv_b, ssm_x_proj,
      ssm_dt_w, ssm_dt_b, alogT, ssm_D, ssm_out_proj)

    logits, emb, s1, s2, s3, s4 = pl.pallas_call(
        _hpc_body,
        out_shape=[
            jax.ShapeDtypeStruct((B, NSIGNS), f32),
            jax.ShapeDtypeStruct((B, DMODEL), f32),
            jax.ShapeDtypeStruct((B, NHS), f32),
            jax.ShapeDtypeStruct((B, NLOC), f32),
            jax.ShapeDtypeStruct((B, NMOV), f32),
            jax.ShapeDtypeStruct((B, NORI), f32),
        ],
        compiler_params=pltpu.CompilerParams(
            vmem_limit_bytes=100 * 1024 * 1024,
        ),
    )(tmean.reshape(B, DMODEL), hpc_w_sign, hpc_b_sign.reshape(1, DMODEL),
      sign_protos, csum.reshape(B, DMODEL), proto_hs, proto_loc,
      proto_mov, proto_ori)
    return (logits, emb, s1, s2, s3, s4)
